# Initial kernel scaffold; baseline (speedup 1.0000x reference)
import jax
import jax.numpy as jnp
from jax.experimental import pallas as pl


def kernel(x, Wr, w1, w2):
    raise NotImplementedError("write your pallas kernel here")


# speedup vs baseline: 1.1476x; 1.1476x over previous
import functools

import jax
import jax.numpy as jnp
from jax.experimental import pallas as pl
from jax.experimental.pallas import tpu as pltpu


def _moe_body(x_ref, wr_ref, w1g_ref, w1u_ref, w2_ref, out_ref, comb_ref):
    e = pl.program_id(0)
    i = pl.program_id(1)

    @pl.when((e == 0) & (i == 0))
    def _router():
        xv = x_ref[...]
        logits = jax.lax.dot_general(
            xv, wr_ref[...], (((1,), (1,)), ((), ())),
            preferred_element_type=jnp.float32)
        m = jnp.max(logits, axis=-1, keepdims=True)
        p = jnp.exp(logits - m)
        p = p / jnp.sum(p, axis=-1, keepdims=True)
        T, E = p.shape
        idxs = jax.lax.broadcasted_iota(jnp.int32, (T, E), 1)
        m1 = jnp.max(p, axis=-1, keepdims=True)
        i1 = jnp.min(jnp.where(p == m1, idxs, E), axis=-1, keepdims=True)
        sel1 = idxs == i1
        p2 = jnp.where(sel1, -jnp.inf, p)
        m2 = jnp.max(p2, axis=-1, keepdims=True)
        i2 = jnp.min(jnp.where(p2 == m2, idxs, E), axis=-1, keepdims=True)
        sel2 = idxs == i2
        denom = m1 + m2
        comb_ref[...] = (jnp.where(sel1, m1 / denom, 0.0)
                         + jnp.where(sel2, m2 / denom, 0.0))
        out_ref[...] = jnp.zeros_like(out_ref)

    xv = x_ref[...]
    gate = jax.lax.dot_general(
        xv, w1g_ref[0], (((1,), (1,)), ((), ())),
        preferred_element_type=jnp.float32)
    up = jax.lax.dot_general(
        xv, w1u_ref[0], (((1,), (1,)), ((), ())),
        preferred_element_type=jnp.float32)
    act = gate * jax.nn.sigmoid(gate) * up
    comb = comb_ref[...]
    lane = jax.lax.broadcasted_iota(jnp.int32, comb.shape, 1)
    cw = jnp.sum(jnp.where(lane == e, comb, 0.0), axis=1, keepdims=True)
    act = act * cw
    out_ref[...] += jax.lax.dot_general(
        act, w2_ref[0], (((1,), (1,)), ((), ())),
        preferred_element_type=jnp.float32)


@functools.partial(jax.jit, static_argnames=("it",))
def _moe(x_flat, Wr, w1g, w1u, w2, it=256):
    T, H = x_flat.shape
    E = Wr.shape[0]
    I = w1g.shape[1]
    ni = I // it
    out = pl.pallas_call(
        _moe_body,
        grid=(E, ni),
        in_specs=[
            pl.BlockSpec((T, H), lambda e, i: (0, 0)),
            pl.BlockSpec((E, H), lambda e, i: (0, 0)),
            pl.BlockSpec((1, it, H), lambda e, i: (e, i, 0)),
            pl.BlockSpec((1, it, H), lambda e, i: (e, i, 0)),
            pl.BlockSpec((1, H, it), lambda e, i: (e, 0, i)),
        ],
        out_specs=pl.BlockSpec((T, H), lambda e, i: (0, 0)),
        out_shape=jax.ShapeDtypeStruct((T, H), jnp.float32),
        scratch_shapes=[pltpu.VMEM((T, E), jnp.float32)],
        compiler_params=pltpu.CompilerParams(
            dimension_semantics=("arbitrary", "arbitrary"),
        ),
    )(x_flat, Wr, w1g, w1u, w2)
    return out


def kernel(x, Wr, w1, w2):
    b, s, h = x.shape
    x_flat = x.reshape(-1, h)
    I = w1.shape[1] // 2
    w1g = w1[:, :I, :]
    w1u = w1[:, I:, :]
    out = _moe(x_flat, Wr, w1g, w1u, w2)
    return out.reshape(b, s, h)
